# baseline (device time: 193640 ns/iter reference)
import jax
import jax.numpy as jnp
from jax import lax
from jax.experimental import pallas as pl
from jax.experimental.pallas import tpu as pltpu

SCALE = 128.0 ** -0.5
D = 128


def _partial_body(rank_ref, q_ref, k_ref, v_ref, onum_ref, l_ref):
    del rank_ref
    n_heads = q_ref.shape[2] // D
    for h in range(n_heads):
        q = q_ref[0, :, h * D:(h + 1) * D].astype(jnp.bfloat16)
        k = k_ref[0, :, h * D:(h + 1) * D].astype(jnp.bfloat16)
        v = v_ref[0, :, h * D:(h + 1) * D].astype(jnp.bfloat16)
        s = lax.dot_general(
            q, k, (((1,), (1,)), ((), ())), preferred_element_type=jnp.float32
        )
        p = jnp.exp(s * SCALE)
        onum = lax.dot_general(
            p.astype(jnp.bfloat16), v, (((1,), (0,)), ((), ())),
            preferred_element_type=jnp.float32,
        )
        onum_ref[0, h] = onum
        l_ref[0, h] = jnp.sum(p, axis=1, keepdims=True)


def _combine_body(
    onum_ref, l_ref, osum_ref, lsum_ref, sbuf, obuf, lbuf, send_sems, recv_sems
):
    my_x = lax.axis_index("x")
    my_y = lax.axis_index("y")
    my_z = lax.axis_index("z")
    partners = (
        (1 - my_x, my_y, my_z),
        (my_x, 1 - my_y, my_z),
        (my_x, my_y, my_z + 1 - 2 * lax.rem(my_z, 2)),
        (my_x, my_y, lax.rem(my_z + 2, 4)),
    )

    barrier = pltpu.get_barrier_semaphore()
    for tgt in partners:
        pl.semaphore_signal(
            barrier, inc=1, device_id=tgt, device_id_type=pl.DeviceIdType.MESH
        )
    pl.semaphore_wait(barrier, len(partners))

    osum_ref[...] = onum_ref[...]
    lsum_ref[...] = l_ref[...]

    for step, tgt in enumerate(partners):
        sbuf[...] = osum_ref[...].astype(jnp.bfloat16)
        o_rdma = pltpu.make_async_remote_copy(
            src_ref=sbuf, dst_ref=obuf.at[step],
            send_sem=send_sems.at[2 * step], recv_sem=recv_sems.at[2 * step],
            device_id=tgt, device_id_type=pl.DeviceIdType.MESH,
        )
        l_rdma = pltpu.make_async_remote_copy(
            src_ref=lsum_ref, dst_ref=lbuf.at[step],
            send_sem=send_sems.at[2 * step + 1],
            recv_sem=recv_sems.at[2 * step + 1],
            device_id=tgt, device_id_type=pl.DeviceIdType.MESH,
        )
        o_rdma.start()
        l_rdma.start()
        o_rdma.wait()
        l_rdma.wait()
        osum_ref[...] = osum_ref[...] + obuf[step].astype(jnp.float32)
        lsum_ref[...] = lsum_ref[...] + lbuf[step]


def kernel(Q, K, V):
    B, SQ, H, d = Q.shape
    KV = K.shape[1]
    KVQ = KV // 4

    rank = (2 * lax.axis_index("x") + lax.axis_index("y")).astype(jnp.int32)
    Q2 = Q.reshape(B, SQ, H * d)
    K2 = K.reshape(B, KV, H * d)
    V2 = V.reshape(B, KV, H * d)

    onum, l = pl.pallas_call(
        _partial_body,
        grid_spec=pltpu.PrefetchScalarGridSpec(
            num_scalar_prefetch=1,
            grid=(B,),
            in_specs=[
                pl.BlockSpec((1, SQ, H * d), lambda bi, rk: (bi, 0, 0)),
                pl.BlockSpec((1, KVQ, H * d), lambda bi, rk: (bi, rk[0], 0)),
                pl.BlockSpec((1, KVQ, H * d), lambda bi, rk: (bi, rk[0], 0)),
            ],
            out_specs=[
                pl.BlockSpec((1, H, SQ, d), lambda bi, rk: (bi, 0, 0, 0)),
                pl.BlockSpec((1, H, SQ, 1), lambda bi, rk: (bi, 0, 0, 0)),
            ],
        ),
        out_shape=[
            jax.ShapeDtypeStruct((B, H, SQ, d), jnp.float32),
            jax.ShapeDtypeStruct((B, H, SQ, 1), jnp.float32),
        ],
    )(rank.reshape(1), Q2, K2, V2)

    l_tile = l.reshape(B, H * SQ)

    osum, lsum = pl.pallas_call(
        _combine_body,
        in_specs=[
            pl.BlockSpec(memory_space=pltpu.VMEM),
            pl.BlockSpec(memory_space=pltpu.VMEM),
        ],
        out_specs=[
            pl.BlockSpec(memory_space=pltpu.VMEM),
            pl.BlockSpec(memory_space=pltpu.VMEM),
        ],
        out_shape=[
            jax.ShapeDtypeStruct((B, H, SQ, d), jnp.float32),
            jax.ShapeDtypeStruct((B, H * SQ), jnp.float32),
        ],
        scratch_shapes=[
            pltpu.VMEM((B, H, SQ, d), jnp.bfloat16),
            pltpu.VMEM((4, B, H, SQ, d), jnp.bfloat16),
            pltpu.VMEM((4, B, H * SQ), jnp.float32),
            pltpu.SemaphoreType.DMA((8,)),
            pltpu.SemaphoreType.DMA((8,)),
        ],
        compiler_params=pltpu.CompilerParams(collective_id=0),
    )(onum, l_tile)

    out = osum / lsum.reshape(B, H, SQ, 1)
    return jnp.transpose(out, (0, 2, 1, 3))


# device time: 68810 ns/iter; 2.8141x vs baseline; 2.8141x over previous
import jax
import jax.numpy as jnp
from jax import lax
from jax.experimental import pallas as pl
from jax.experimental.pallas import tpu as pltpu

SCALE = 128.0 ** -0.5


def _partial_body(q_ref, k_ref, v_ref, onum_ref, l_ref):
    n_heads = q_ref.shape[1]
    for h in range(n_heads):
        q = q_ref[0, h]
        k = k_ref[0, h]
        v = v_ref[0, h]
        s = lax.dot_general(
            q, k, (((1,), (1,)), ((), ())), preferred_element_type=jnp.float32
        )
        p = jnp.exp(s * SCALE)
        onum = lax.dot_general(
            p.astype(jnp.bfloat16), v, (((1,), (0,)), ((), ())),
            preferred_element_type=jnp.float32,
        )
        onum_ref[0, h] = onum
        l_ref[0, h] = jnp.sum(p, axis=1, keepdims=True)


def _combine_body(
    onum_ref, l_ref, osum_ref, lsum_ref,
    sbuf, obuf, lbuf, os_sems, or_sems, ls_sems, lr_sems,
):
    my_x = lax.axis_index("x")
    my_y = lax.axis_index("y")
    my_z = lax.axis_index("z")
    plane = (
        (1 - my_x, my_y, my_z),
        (my_x, 1 - my_y, my_z),
        (1 - my_x, 1 - my_y, my_z),
    )
    column = (
        (my_x, my_y, lax.rem(my_z + 1, 4)),
        (my_x, my_y, lax.rem(my_z + 2, 4)),
        (my_x, my_y, lax.rem(my_z + 3, 4)),
    )
    partners = plane + column

    barrier = pltpu.get_barrier_semaphore()
    for tgt in partners:
        pl.semaphore_signal(
            barrier, inc=1, device_id=tgt, device_id_type=pl.DeviceIdType.MESH
        )
    pl.semaphore_wait(barrier, len(partners))

    lsum_ref[...] = l_ref[...]

    for phase, group in enumerate((plane, column)):
        if phase == 0:
            sbuf[...] = onum_ref[...].astype(jnp.bfloat16)
        else:
            sbuf[...] = osum_ref[...].astype(jnp.bfloat16)
        rdmas = []
        for i, tgt in enumerate(group):
            slot = 3 * phase + i
            o_rdma = pltpu.make_async_remote_copy(
                src_ref=sbuf, dst_ref=obuf.at[slot],
                send_sem=os_sems.at[slot], recv_sem=or_sems.at[slot],
                device_id=tgt, device_id_type=pl.DeviceIdType.MESH,
            )
            l_rdma = pltpu.make_async_remote_copy(
                src_ref=lsum_ref, dst_ref=lbuf.at[slot],
                send_sem=ls_sems.at[slot], recv_sem=lr_sems.at[slot],
                device_id=tgt, device_id_type=pl.DeviceIdType.MESH,
            )
            o_rdma.start()
            l_rdma.start()
            rdmas.append((o_rdma, l_rdma))
        for o_rdma, l_rdma in rdmas:
            o_rdma.wait()
            l_rdma.wait()
        base = onum_ref[...] if phase == 0 else osum_ref[...]
        osum_ref[...] = (
            base
            + obuf[3 * phase + 0].astype(jnp.float32)
            + obuf[3 * phase + 1].astype(jnp.float32)
            + obuf[3 * phase + 2].astype(jnp.float32)
        )
        lsum_ref[...] = (
            lsum_ref[...]
            + lbuf[3 * phase + 0]
            + lbuf[3 * phase + 1]
            + lbuf[3 * phase + 2]
        )


def kernel(Q, K, V):
    B, SQ, H, D = Q.shape
    KV = K.shape[1]
    KVQ = KV // 4

    rank = 2 * lax.axis_index("x") + lax.axis_index("y")
    Kq = lax.dynamic_slice(K, (0, rank * KVQ, 0, 0), (B, KVQ, H, D))
    Vq = lax.dynamic_slice(V, (0, rank * KVQ, 0, 0), (B, KVQ, H, D))
    Kt = jnp.transpose(Kq.astype(jnp.bfloat16), (0, 2, 1, 3))
    Vt = jnp.transpose(Vq.astype(jnp.bfloat16), (0, 2, 1, 3))
    Qt = jnp.transpose(Q, (0, 2, 1, 3)).astype(jnp.bfloat16)

    onum, l = pl.pallas_call(
        _partial_body,
        grid=(B,),
        in_specs=[
            pl.BlockSpec((1, H, SQ, D), lambda bi: (bi, 0, 0, 0)),
            pl.BlockSpec((1, H, KVQ, D), lambda bi: (bi, 0, 0, 0)),
            pl.BlockSpec((1, H, KVQ, D), lambda bi: (bi, 0, 0, 0)),
        ],
        out_specs=[
            pl.BlockSpec((1, H, SQ, D), lambda bi: (bi, 0, 0, 0)),
            pl.BlockSpec((1, H, SQ, 1), lambda bi: (bi, 0, 0, 0)),
        ],
        out_shape=[
            jax.ShapeDtypeStruct((B, H, SQ, D), jnp.float32),
            jax.ShapeDtypeStruct((B, H, SQ, 1), jnp.float32),
        ],
    )(Qt, Kt, Vt)

    l_tile = l.reshape(B, H * SQ)

    osum, lsum = pl.pallas_call(
        _combine_body,
        in_specs=[
            pl.BlockSpec(memory_space=pltpu.VMEM),
            pl.BlockSpec(memory_space=pltpu.VMEM),
        ],
        out_specs=[
            pl.BlockSpec(memory_space=pltpu.VMEM),
            pl.BlockSpec(memory_space=pltpu.VMEM),
        ],
        out_shape=[
            jax.ShapeDtypeStruct((B, H, SQ, D), jnp.float32),
            jax.ShapeDtypeStruct((B, H * SQ), jnp.float32),
        ],
        scratch_shapes=[
            pltpu.VMEM((B, H, SQ, D), jnp.bfloat16),
            pltpu.VMEM((6, B, H, SQ, D), jnp.bfloat16),
            pltpu.VMEM((6, B, H * SQ), jnp.float32),
            pltpu.SemaphoreType.DMA((6,)),
            pltpu.SemaphoreType.DMA((6,)),
            pltpu.SemaphoreType.DMA((6,)),
            pltpu.SemaphoreType.DMA((6,)),
        ],
        compiler_params=pltpu.CompilerParams(collective_id=0),
    )(onum, l_tile)

    out = osum / lsum.reshape(B, H, SQ, 1)
    return jnp.transpose(out, (0, 2, 1, 3))


# device time: 52073 ns/iter; 3.7186x vs baseline; 1.3214x over previous
import jax
import jax.numpy as jnp
from jax import lax
from jax.experimental import pallas as pl
from jax.experimental.pallas import tpu as pltpu

SCALE = 128.0 ** -0.5


def _partial_body(q_ref, k_ref, v_ref, onum_ref, l_ref):
    q = q_ref[0]
    k = k_ref[0]
    v = v_ref[0]
    s = lax.dot_general(
        q, k, (((2,), (2,)), ((0,), (0,))),
        preferred_element_type=jnp.float32,
    )
    p = jnp.exp(s * SCALE)
    onum_ref[0] = lax.dot_general(
        p.astype(jnp.bfloat16), v, (((2,), (1,)), ((0,), (0,))),
        preferred_element_type=jnp.float32,
    )
    l_ref[0] = jnp.sum(p, axis=2, keepdims=True)


def _combine_body(
    onum_ref, l_ref, osum_ref, lsum_ref,
    sbuf, obuf, lbuf, os_sems, or_sems, ls_sems, lr_sems,
):
    my_x = lax.axis_index("x")
    my_y = lax.axis_index("y")
    my_z = lax.axis_index("z")
    plane = (
        (1 - my_x, my_y, my_z),
        (my_x, 1 - my_y, my_z),
        (1 - my_x, 1 - my_y, my_z),
    )
    column = (
        (my_x, my_y, lax.rem(my_z + 1, 4)),
        (my_x, my_y, lax.rem(my_z + 2, 4)),
        (my_x, my_y, lax.rem(my_z + 3, 4)),
    )
    partners = plane + column

    barrier = pltpu.get_barrier_semaphore()
    for tgt in partners:
        pl.semaphore_signal(
            barrier, inc=1, device_id=tgt, device_id_type=pl.DeviceIdType.MESH
        )
    pl.semaphore_wait(barrier, len(partners))

    lsum_ref[...] = l_ref[...]

    for phase, group in enumerate((plane, column)):
        if phase == 0:
            sbuf[...] = onum_ref[...].astype(jnp.bfloat16)
        else:
            sbuf[...] = osum_ref[...].astype(jnp.bfloat16)
        rdmas = []
        for i, tgt in enumerate(group):
            slot = 3 * phase + i
            o_rdma = pltpu.make_async_remote_copy(
                src_ref=sbuf, dst_ref=obuf.at[slot],
                send_sem=os_sems.at[slot], recv_sem=or_sems.at[slot],
                device_id=tgt, device_id_type=pl.DeviceIdType.MESH,
            )
            l_rdma = pltpu.make_async_remote_copy(
                src_ref=lsum_ref, dst_ref=lbuf.at[slot],
                send_sem=ls_sems.at[slot], recv_sem=lr_sems.at[slot],
                device_id=tgt, device_id_type=pl.DeviceIdType.MESH,
            )
            o_rdma.start()
            l_rdma.start()
            rdmas.append((o_rdma, l_rdma))
        for o_rdma, l_rdma in rdmas:
            o_rdma.wait()
            l_rdma.wait()
        base = onum_ref[...] if phase == 0 else osum_ref[...]
        osum_ref[...] = (
            base
            + obuf[3 * phase + 0].astype(jnp.float32)
            + obuf[3 * phase + 1].astype(jnp.float32)
            + obuf[3 * phase + 2].astype(jnp.float32)
        )
        lsum_ref[...] = (
            lsum_ref[...]
            + lbuf[3 * phase + 0]
            + lbuf[3 * phase + 1]
            + lbuf[3 * phase + 2]
        )


def kernel(Q, K, V):
    B, SQ, H, D = Q.shape
    KV = K.shape[1]
    KVQ = KV // 4

    rank = 2 * lax.axis_index("x") + lax.axis_index("y")
    Kq = lax.dynamic_slice(K, (0, rank * KVQ, 0, 0), (B, KVQ, H, D))
    Vq = lax.dynamic_slice(V, (0, rank * KVQ, 0, 0), (B, KVQ, H, D))
    Kt = jnp.transpose(Kq.astype(jnp.bfloat16), (0, 2, 1, 3))
    Vt = jnp.transpose(Vq.astype(jnp.bfloat16), (0, 2, 1, 3))
    Qt = jnp.transpose(Q, (0, 2, 1, 3)).astype(jnp.bfloat16)

    onum, l = pl.pallas_call(
        _partial_body,
        grid=(B,),
        in_specs=[
            pl.BlockSpec((1, H, SQ, D), lambda bi: (bi, 0, 0, 0)),
            pl.BlockSpec((1, H, KVQ, D), lambda bi: (bi, 0, 0, 0)),
            pl.BlockSpec((1, H, KVQ, D), lambda bi: (bi, 0, 0, 0)),
        ],
        out_specs=[
            pl.BlockSpec((1, H, SQ, D), lambda bi: (bi, 0, 0, 0)),
            pl.BlockSpec((1, H, SQ, 1), lambda bi: (bi, 0, 0, 0)),
        ],
        out_shape=[
            jax.ShapeDtypeStruct((B, H, SQ, D), jnp.float32),
            jax.ShapeDtypeStruct((B, H, SQ, 1), jnp.float32),
        ],
    )(Qt, Kt, Vt)

    l_tile = l.reshape(B, H * SQ)

    osum, lsum = pl.pallas_call(
        _combine_body,
        in_specs=[
            pl.BlockSpec(memory_space=pltpu.VMEM),
            pl.BlockSpec(memory_space=pltpu.VMEM),
        ],
        out_specs=[
            pl.BlockSpec(memory_space=pltpu.VMEM),
            pl.BlockSpec(memory_space=pltpu.VMEM),
        ],
        out_shape=[
            jax.ShapeDtypeStruct((B, H, SQ, D), jnp.float32),
            jax.ShapeDtypeStruct((B, H * SQ), jnp.float32),
        ],
        scratch_shapes=[
            pltpu.VMEM((B, H, SQ, D), jnp.bfloat16),
            pltpu.VMEM((6, B, H, SQ, D), jnp.bfloat16),
            pltpu.VMEM((6, B, H * SQ), jnp.float32),
            pltpu.SemaphoreType.DMA((6,)),
            pltpu.SemaphoreType.DMA((6,)),
            pltpu.SemaphoreType.DMA((6,)),
            pltpu.SemaphoreType.DMA((6,)),
        ],
        compiler_params=pltpu.CompilerParams(collective_id=0),
    )(onum, l_tile)

    out = osum / lsum.reshape(B, H, SQ, 1)
    return jnp.transpose(out, (0, 2, 1, 3))
